# baseline (speedup 1.0000x reference)
import functools

import jax
import jax.numpy as jnp
from jax import lax
from jax.experimental import pallas as pl
from jax.experimental.pallas import tpu as pltpu
from jax.experimental.pallas import tpu_sc as plsc

N = 10000
E = 320000
D = 16
N_PAD = 10240
NC = 2
NS = 16
NW = NC * NS
E_PER_W = E // NW
WIN = 2000
CHUNKS = [(j * 128, 128) for j in range(15)] + [(1920, 80)]
N_WIN = E_PER_W // WIN
ROWS_PER_S = N_PAD // NS

_MESH = dict(core_axis_name="c", subcore_axis_name="s",
             num_cores=NC, num_subcores=NS)


def _sc_partials(ei, attr, zrow, ones):
  @functools.partial(
      pl.kernel,
      out_type=[
          jax.ShapeDtypeStruct((NC, N_PAD, D), jnp.float32),
          jax.ShapeDtypeStruct((NC, N_PAD, D), jnp.float32),
      ],
      mesh=plsc.VectorSubcoreMesh(**_MESH),
      compiler_params=pltpu.CompilerParams(use_tc_tiling_on_sc=False),
      scratch_types=[
          pltpu.VMEM_SHARED((N_PAD, D), jnp.float32),
          pltpu.VMEM_SHARED((N_PAD, D), jnp.float32),
          pltpu.VMEM((2, 2, WIN), jnp.int32),
          pltpu.VMEM((2, WIN, D), jnp.float32),
          pltpu.VMEM((WIN, D), jnp.float32),
          pltpu.SemaphoreType.DMA,
          pltpu.SemaphoreType.DMA,
      ],
  )
  def k(ei_hbm, attr_hbm, zrow_hbm, ones_hbm, psum_hbm, pcnt_hbm,
        acc, cnt, idx_v, attr_v, ones_v, sem_in, sem_sc):
    c = lax.axis_index("c")
    s = lax.axis_index("s")
    wid = s * NC + c
    rbase = s * ROWS_PER_S

    pltpu.sync_copy(zrow_hbm, acc.at[pl.ds(rbase, ROWS_PER_S)])
    pltpu.sync_copy(zrow_hbm, cnt.at[pl.ds(rbase, ROWS_PER_S)])
    pltpu.sync_copy(ones_hbm, ones_v)
    plsc.subcore_barrier()

    def fire_in(w):
      b = w % 2
      ebase = pl.multiple_of(wid * E_PER_W + w * WIN, WIN)
      return [
          pltpu.async_copy(ei_hbm.at[:, pl.ds(ebase, WIN)], idx_v.at[b],
                           sem_in),
          pltpu.async_copy(attr_hbm.at[pl.ds(ebase, WIN)], attr_v.at[b],
                           sem_in),
      ]

    in_descs = {0: fire_in(0)}
    for w in range(N_WIN):
      b = w % 2
      for d in in_descs.pop(w):
        d.wait()
      if w + 1 < N_WIN:
        in_descs[w + 1] = fire_in(w + 1)

      sc_descs = []
      for off, ln in CHUNKS:
        sc_descs.append(
            pltpu.async_copy(attr_v.at[b, pl.ds(off, ln)],
                             acc.at[idx_v.at[b, 1, pl.ds(off, ln)]],
                             sem_sc, add=True))
        sc_descs.append(
            pltpu.async_copy(ones_v.at[pl.ds(off, ln)],
                             cnt.at[idx_v.at[b, 1, pl.ds(off, ln)]],
                             sem_sc, add=True))
      for d in sc_descs:
        d.wait()

    plsc.subcore_barrier()

    pltpu.sync_copy(acc.at[pl.ds(rbase, ROWS_PER_S)],
                    psum_hbm.at[c, pl.ds(rbase, ROWS_PER_S)])
    pltpu.sync_copy(cnt.at[pl.ds(rbase, ROWS_PER_S)],
                    pcnt_hbm.at[c, pl.ds(rbase, ROWS_PER_S)])

  return k(ei, attr, zrow, ones)


def _combine(psum, pcnt):
  def body(ps_ref, pc_ref, out_ref):
    sums = ps_ref[0] + ps_ref[1]
    counts = pc_ref[0] + pc_ref[1]
    out_ref[...] = (sums / jnp.clip(counts, 1.0, None))[:N]

  return pl.pallas_call(
      body,
      out_shape=jax.ShapeDtypeStruct((N, D), jnp.float32),
  )(psum, pcnt)


@jax.jit
def kernel(x, edge_index, edge_attr):
  del x
  ei = edge_index.astype(jnp.int32)
  zrow = jnp.zeros((ROWS_PER_S, D), jnp.float32)
  ones = jnp.ones((WIN, D), jnp.float32)
  psum, pcnt = _sc_partials(ei, edge_attr, zrow, ones)
  return _combine(psum, pcnt)


# speedup vs baseline: 1.1051x; 1.1051x over previous
import functools

import jax
import jax.numpy as jnp
from jax import lax
from jax.experimental import pallas as pl
from jax.experimental.pallas import tpu as pltpu
from jax.experimental.pallas import tpu_sc as plsc

N = 10000
E = 320000
D = 16
N_PAD = 10240
NC = 2
NS = 16
NW = NC * NS
E_PER_W = E // NW
WIN = 2000
CHUNKS = [(j * 128, 128) for j in range(15)] + [(1920, 80)]
N_WIN = E_PER_W // WIN
ROWS_PER_S = N_PAD // NS

_MESH = dict(core_axis_name="c", subcore_axis_name="s",
             num_cores=NC, num_subcores=NS)


def _sc_sums(ei, attr, zrow):
  @functools.partial(
      pl.kernel,
      out_type=jax.ShapeDtypeStruct((NC, N_PAD, D), jnp.float32),
      mesh=plsc.VectorSubcoreMesh(**_MESH),
      compiler_params=pltpu.CompilerParams(use_tc_tiling_on_sc=False),
      scratch_types=[
          pltpu.VMEM_SHARED((N_PAD, D), jnp.float32),
          pltpu.VMEM((2, 2, WIN), jnp.int32),
          pltpu.VMEM((2, WIN, D), jnp.float32),
          pltpu.SemaphoreType.DMA,
          pltpu.SemaphoreType.DMA,
      ],
  )
  def k(ei_hbm, attr_hbm, zrow_hbm, psum_hbm, acc, idx_v, attr_v,
        sem_in, sem_sc):
    c = lax.axis_index("c")
    s = lax.axis_index("s")
    wid = s * NC + c
    rbase = s * ROWS_PER_S

    pltpu.sync_copy(zrow_hbm, acc.at[pl.ds(rbase, ROWS_PER_S)])
    plsc.subcore_barrier()

    def fire_in(w):
      b = w % 2
      ebase = pl.multiple_of(wid * E_PER_W + w * WIN, WIN)
      return [
          pltpu.async_copy(ei_hbm.at[:, pl.ds(ebase, WIN)], idx_v.at[b],
                           sem_in),
          pltpu.async_copy(attr_hbm.at[pl.ds(ebase, WIN)], attr_v.at[b],
                           sem_in),
      ]

    in_descs = {0: fire_in(0)}
    for w in range(N_WIN):
      b = w % 2
      for d in in_descs.pop(w):
        d.wait()
      if w + 1 < N_WIN:
        in_descs[w + 1] = fire_in(w + 1)

      sc_descs = [
          pltpu.async_copy(attr_v.at[b, pl.ds(off, ln)],
                           acc.at[idx_v.at[b, 1, pl.ds(off, ln)]],
                           sem_sc, add=True)
          for off, ln in CHUNKS
      ]
      for d in sc_descs:
        d.wait()

    plsc.subcore_barrier()

    pltpu.sync_copy(acc.at[pl.ds(rbase, ROWS_PER_S)],
                    psum_hbm.at[c, pl.ds(rbase, ROWS_PER_S)])

  return k(ei, attr, zrow)


def _sc_counts(ei):
  @functools.partial(
      pl.kernel,
      out_type=jax.ShapeDtypeStruct((NW, N_PAD), jnp.float32),
      mesh=plsc.VectorSubcoreMesh(**_MESH),
      compiler_params=pltpu.CompilerParams(
          use_tc_tiling_on_sc=False, needs_layout_passes=False),
      scratch_types=[
          pltpu.VMEM((2, 2, WIN), jnp.int32),
          pltpu.VMEM((N_PAD,), jnp.float32),
          pltpu.SemaphoreType.DMA,
      ],
  )
  def k(ei_hbm, pcnt_hbm, idxf_v, hist, sem_in):
    c = lax.axis_index("c")
    s = lax.axis_index("s")
    wid = s * NC + c
    ones16 = jnp.ones((16,), jnp.float32)
    zeros16 = jnp.zeros((16,), jnp.float32)

    def zbody(g, carry):
      hist[pl.ds(g * 16, 16)] = zeros16
      return carry
    lax.fori_loop(0, N_PAD // 16, zbody, 0, unroll=8)

    def fire(w):
      b = w % 2
      ebase = pl.multiple_of(wid * E_PER_W + w * WIN, WIN)
      return pltpu.async_copy(ei_hbm.at[:, pl.ds(ebase, WIN)], idxf_v.at[b],
                              sem_in)

    descs = {0: fire(0)}
    for w in range(N_WIN):
      b = w % 2
      descs.pop(w).wait()
      if w + 1 < N_WIN:
        descs[w + 1] = fire(w + 1)

      def hbody(g, carry, b=b):
        iv = idxf_v[b, 1, pl.ds(g * 16, 16)]
        plsc.addupdate_scatter(hist, [iv], ones16)
        return carry
      lax.fori_loop(0, WIN // 16, hbody, 0, unroll=5)

    pltpu.sync_copy(hist, pcnt_hbm.at[wid])

  return k(ei)


def _combine(psum, pcnt):
  def body(ps_ref, pc_ref, out_ref):
    sums = ps_ref[0] + ps_ref[1]
    ones = jnp.ones((NW, 1), jnp.float32)
    counts = jax.lax.dot_general(
        pc_ref[...], ones, (((0,), (0,)), ((), ())),
        preferred_element_type=jnp.float32)
    out_ref[...] = (sums / jnp.clip(counts, 1.0, None))[:N]

  return pl.pallas_call(
      body,
      out_shape=jax.ShapeDtypeStruct((N, D), jnp.float32),
  )(psum, pcnt)


@jax.jit
def kernel(x, edge_index, edge_attr):
  del x
  ei = edge_index.astype(jnp.int32)
  zrow = jnp.zeros((ROWS_PER_S, D), jnp.float32)
  psum = _sc_sums(ei, edge_attr, zrow)
  pcnt = _sc_counts(ei)
  return _combine(psum, pcnt)
